# baseline (speedup 1.0000x reference)
import functools

import jax
import jax.numpy as jnp
import numpy as np
from jax import lax
from jax.experimental import pallas as pl
from jax.experimental.pallas import tpu as pltpu
from jax.experimental.pallas import tpu_sc as plsc

N = 10000
E = 320000
D = 128
NG = 32
NHID = 256
NOUT = 768

NC = 2
NS = 16
NW = NC * NS
CHUNK = 128
TOTC = 2560
KA = 56
KB = 104
KMAX = 104
EPAD = TOTC * CHUNK
NPAD = 10240
ROWS_PER_TILE = NPAD // NS
INV_SQRT2 = float(1.0 / np.sqrt(2.0))


_sc_kernel_cache = {}


def _sc_deg_body(dst_hbm, zeros_hbm, ones_hbm, out_hbm, didx, ones_v, acc):
    cid = lax.axis_index("c")
    sid = lax.axis_index("s")
    base = jnp.where(cid == 0, sid * KA, NS * KA + sid * KB)
    cnt = jnp.where(cid == 0, KA, KB)
    pltpu.sync_copy(zeros_hbm.at[pl.ds(sid * ROWS_PER_TILE, ROWS_PER_TILE)],
                    acc.at[pl.ds(sid * ROWS_PER_TILE, ROWS_PER_TILE)])
    pltpu.sync_copy(dst_hbm.at[pl.ds(base, KMAX)], didx)
    pltpu.sync_copy(ones_hbm, ones_v)
    plsc.subcore_barrier()

    def body(j, carry):
        pltpu.sync_copy(ones_v, acc.at[didx.at[j]], add=True)
        return carry

    lax.fori_loop(0, cnt, body, 0)
    plsc.subcore_barrier()
    pltpu.sync_copy(acc.at[pl.ds(sid * ROWS_PER_TILE, ROWS_PER_TILE)],
                    out_hbm.at[cid, pl.ds(sid * ROWS_PER_TILE, ROWS_PER_TILE)])


def _run_deg(dst3, zeros_big, ones_big):
    if "deg" not in _sc_kernel_cache:
        mesh = plsc.VectorSubcoreMesh(core_axis_name="c", subcore_axis_name="s")
        _sc_kernel_cache["deg"] = functools.partial(
            pl.kernel,
            out_type=jax.ShapeDtypeStruct((NC, NPAD, D), jnp.float32),
            mesh=mesh,
            scratch_types=[
                pltpu.VMEM((KMAX, CHUNK), jnp.int32),
                pltpu.VMEM((CHUNK, D), jnp.float32),
                pltpu.VMEM_SHARED((NPAD, D), jnp.float32),
            ],
        )(_sc_deg_body)
    return _sc_kernel_cache["deg"](dst3, zeros_big, ones_big)


def _sc_scatter_body(y_hbm, src_hbm, dst_hbm, zeros_hbm, out_hbm,
                     sidx, didx, gbuf, acc, gsem):
    cid = lax.axis_index("c")
    sid = lax.axis_index("s")
    base = jnp.where(cid == 0, sid * KA, NS * KA + sid * KB)
    cnt = jnp.where(cid == 0, KA, KB)
    pltpu.sync_copy(zeros_hbm.at[pl.ds(sid * ROWS_PER_TILE, ROWS_PER_TILE)],
                    acc.at[pl.ds(sid * ROWS_PER_TILE, ROWS_PER_TILE)])
    pltpu.sync_copy(src_hbm.at[pl.ds(base, KMAX)], sidx)
    pltpu.sync_copy(dst_hbm.at[pl.ds(base, KMAX)], didx)
    plsc.subcore_barrier()

    def body(j, carry):
        pltpu.async_copy(y_hbm.at[sidx.at[j]], gbuf, gsem).wait()
        pltpu.sync_copy(gbuf, acc.at[didx.at[j]], add=True)
        return carry

    lax.fori_loop(0, cnt, body, 0)
    plsc.subcore_barrier()
    pltpu.sync_copy(acc.at[pl.ds(sid * ROWS_PER_TILE, ROWS_PER_TILE)],
                    out_hbm.at[cid, pl.ds(sid * ROWS_PER_TILE, ROWS_PER_TILE)])


def _run_scatter(y, src3, dst3, zeros_big):
    if "scatter" not in _sc_kernel_cache:
        mesh = plsc.VectorSubcoreMesh(core_axis_name="c", subcore_axis_name="s")
        _sc_kernel_cache["scatter"] = functools.partial(
            pl.kernel,
            out_type=jax.ShapeDtypeStruct((NC, NPAD, D), jnp.float32),
            mesh=mesh,
            scratch_types=[
                pltpu.VMEM((KMAX, CHUNK), jnp.int32),
                pltpu.VMEM((KMAX, CHUNK), jnp.int32),
                pltpu.VMEM((CHUNK, D), jnp.float32),
                pltpu.VMEM_SHARED((NPAD, D), jnp.float32),
                pltpu.SemaphoreType.DMA,
            ],
        )(_sc_scatter_body)
    return _sc_kernel_cache["scatter"](y, src3, dst3, zeros_big)


def _prep_body(degp_ref, x_ref, w_ref, dis_ref, y_ref):
    deg = degp_ref[0, :N, 0:1] + degp_ref[1, :N, 0:1] + 1.0
    dis = lax.rsqrt(deg)
    dis_ref[...] = dis
    xw = jnp.dot(x_ref[...], w_ref[...], preferred_element_type=jnp.float32)
    y_ref[...] = dis * xw


_prep_kernel = pl.pallas_call(
    _prep_body,
    out_shape=(jax.ShapeDtypeStruct((N, 1), jnp.float32),
               jax.ShapeDtypeStruct((N, D), jnp.float32)),
)


def _bn_relu(zp_ref, y_ref, dis_ref, b_ref, g_ref, be_ref):
    z = zp_ref[0, :N, :] + zp_ref[1, :N, :]
    t = dis_ref[...] * (z + y_ref[...]) + b_ref[...]
    m = jnp.mean(t, axis=0, keepdims=True)
    v = jnp.mean((t - m) * (t - m), axis=0, keepdims=True)
    t = (t - m) * lax.rsqrt(v + 1e-5) * g_ref[...] + be_ref[...]
    return jnp.maximum(t, 0.0)


def _mid_even_body(zp_ref, y_ref, dis_ref, b_ref, g_ref, be_ref, wn_ref,
                   ynext_ref):
    t = _bn_relu(zp_ref, y_ref, dis_ref, b_ref, g_ref, be_ref)
    ynext_ref[...] = dis_ref[...] * jnp.dot(
        t, wn_ref[...], preferred_element_type=jnp.float32)


_mid_even_kernel = pl.pallas_call(
    _mid_even_body,
    out_shape=jax.ShapeDtypeStruct((N, D), jnp.float32),
)


def _mid_odd_body(zp_ref, y_ref, dis_ref, b_ref, g_ref, be_ref, hin_ref,
                  wn_ref, hnew_ref, ynext_ref):
    t = _bn_relu(zp_ref, y_ref, dis_ref, b_ref, g_ref, be_ref)
    h = (t + hin_ref[...]) * INV_SQRT2
    hnew_ref[...] = h
    ynext_ref[...] = dis_ref[...] * jnp.dot(
        h, wn_ref[...], preferred_element_type=jnp.float32)


_mid_odd_kernel = pl.pallas_call(
    _mid_odd_body,
    out_shape=(jax.ShapeDtypeStruct((N, D), jnp.float32),
               jax.ShapeDtypeStruct((N, D), jnp.float32)),
)


def _final_body(zp_ref, y_ref, dis_ref, b_ref, g_ref, be_ref, hin_ref,
                batch_ref, wh1_ref, bh1_ref, wh2_ref, bh2_ref, out_ref):
    t = _bn_relu(zp_ref, y_ref, dis_ref, b_ref, g_ref, be_ref)
    h = (t + hin_ref[...]) * INV_SQRT2
    seg = batch_ref[...]
    gid = lax.broadcasted_iota(jnp.int32, (1, NG), 1)
    p = (seg == gid).astype(jnp.float32)
    dn = (((0,), (0,)), ((), ()))
    sums = lax.dot_general(p, h, dn, preferred_element_type=jnp.float32)
    cnt = lax.dot_general(p, jnp.ones((N, 1), jnp.float32), dn,
                          preferred_element_type=jnp.float32)
    pooled = sums / jnp.maximum(cnt, 1.0)
    z1 = jnp.dot(pooled, wh1_ref[...], preferred_element_type=jnp.float32)
    z1 = jnp.maximum(z1 + bh1_ref[...], 0.0)
    z2 = jnp.dot(z1, wh2_ref[...], preferred_element_type=jnp.float32)
    out_ref[...] = z2 + bh2_ref[...]


_final_kernel = pl.pallas_call(
    _final_body,
    out_shape=jax.ShapeDtypeStruct((NG, NOUT), jnp.float32),
)


def kernel(x, edge_index, batch, W1s, b1s, g1s, be1s, W2s, b2s, g2s, be2s,
           Wh1, bh1, Wh2, bh2):
    src = edge_index[0]
    dst = edge_index[1]
    pad = EPAD - E
    srcp = jnp.concatenate([src, jnp.zeros((pad,), jnp.int32)])
    dstp = jnp.concatenate([dst, jnp.full((pad,), NPAD - 1, jnp.int32)])
    src3 = srcp.reshape(TOTC, CHUNK)
    dst3 = dstp.reshape(TOTC, CHUNK)

    zeros_big = jnp.zeros((NPAD, D), jnp.float32)
    ones_big = jnp.ones((CHUNK, D), jnp.float32)
    batch2d = batch.reshape(N, 1)

    ws = [W1s[0], W2s[0], W1s[1], W2s[1], W1s[2], W2s[2]]
    bs = [b1s[0], b2s[0], b1s[1], b2s[1], b1s[2], b2s[2]]
    gs = [g1s[0], g2s[0], g1s[1], g2s[1], g1s[2], g2s[2]]
    bes = [be1s[0], be2s[0], be1s[1], be2s[1], be1s[2], be2s[2]]
    bs = [b.reshape(1, D) for b in bs]
    gs = [g.reshape(1, D) for g in gs]
    bes = [b.reshape(1, D) for b in bes]

    degp = _run_deg(dst3, zeros_big, ones_big)
    dis, y = _prep_kernel(degp, x, ws[0])

    h_in = x
    out = None
    for i in range(6):
        zp = _run_scatter(y, src3, dst3, zeros_big)
        if i == 5:
            out = _final_kernel(zp, y, dis, bs[i], gs[i], bes[i], h_in,
                                batch2d, Wh1, bh1.reshape(1, NHID), Wh2,
                                bh2.reshape(1, NOUT))
        elif i % 2 == 0:
            y = _mid_even_kernel(zp, y, dis, bs[i], gs[i], bes[i], ws[i + 1])
        else:
            h_in, y = _mid_odd_kernel(zp, y, dis, bs[i], gs[i], bes[i], h_in,
                                      ws[i + 1])
    return out


# speedup vs baseline: 1.1561x; 1.1561x over previous
import functools

import jax
import jax.numpy as jnp
import numpy as np
from jax import lax
from jax.experimental import pallas as pl
from jax.experimental.pallas import tpu as pltpu
from jax.experimental.pallas import tpu_sc as plsc

N = 10000
E = 320000
D = 128
NG = 32
NHID = 256
NOUT = 768

NC = 2
NS = 16
NW = NC * NS
CHUNK = 128
TOTC = 2560
KA = 104
KB = 56
KMAX = 104
EPAD = TOTC * CHUNK
NPAD = 10240
ROWS_PER_TILE = NPAD // NS
INV_SQRT2 = float(1.0 / np.sqrt(2.0))


_sc_kernel_cache = {}


def _sc_deg_body(dst_hbm, zeros_hbm, ones_hbm, out_hbm, didx, ones_v, acc):
    cid = lax.axis_index("c")
    sid = lax.axis_index("s")
    base = jnp.where(cid == 0, sid * KA, NS * KA + sid * KB)
    cnt = jnp.where(cid == 0, KA, KB)
    pltpu.sync_copy(zeros_hbm.at[pl.ds(sid * ROWS_PER_TILE, ROWS_PER_TILE)],
                    acc.at[pl.ds(sid * ROWS_PER_TILE, ROWS_PER_TILE)])
    pltpu.sync_copy(dst_hbm.at[pl.ds(base, KMAX)], didx)
    pltpu.sync_copy(ones_hbm, ones_v)
    plsc.subcore_barrier()

    def body(j, carry):
        pltpu.sync_copy(ones_v, acc.at[didx.at[j]], add=True)
        return carry

    lax.fori_loop(0, cnt, body, 0)
    plsc.subcore_barrier()
    pltpu.sync_copy(acc.at[pl.ds(sid * ROWS_PER_TILE, ROWS_PER_TILE)],
                    out_hbm.at[cid, pl.ds(sid * ROWS_PER_TILE, ROWS_PER_TILE)])


def _run_deg(dst3, zeros_big, ones_big):
    if "deg" not in _sc_kernel_cache:
        mesh = plsc.VectorSubcoreMesh(core_axis_name="c", subcore_axis_name="s")
        _sc_kernel_cache["deg"] = functools.partial(
            pl.kernel,
            out_type=jax.ShapeDtypeStruct((NC, NPAD, D), jnp.float32),
            mesh=mesh,
            scratch_types=[
                pltpu.VMEM((KMAX, CHUNK), jnp.int32),
                pltpu.VMEM((CHUNK, D), jnp.float32),
                pltpu.VMEM_SHARED((NPAD, D), jnp.float32),
            ],
        )(_sc_deg_body)
    return _sc_kernel_cache["deg"](dst3, zeros_big, ones_big)


def _sc_scatter_body(y_hbm, src_hbm, dst_hbm, zeros_hbm, out_hbm,
                     sidx, didx, gbuf, acc, gsem):
    cid = lax.axis_index("c")
    sid = lax.axis_index("s")
    base = jnp.where(cid == 0, sid * KA, NS * KA + sid * KB)
    cnt = jnp.where(cid == 0, KA, KB)
    pltpu.sync_copy(zeros_hbm.at[pl.ds(sid * ROWS_PER_TILE, ROWS_PER_TILE)],
                    acc.at[pl.ds(sid * ROWS_PER_TILE, ROWS_PER_TILE)])
    pltpu.sync_copy(src_hbm.at[pl.ds(base, KMAX)], sidx)
    pltpu.sync_copy(dst_hbm.at[pl.ds(base, KMAX)], didx)
    plsc.subcore_barrier()

    def body(j, carry):
        pltpu.async_copy(y_hbm.at[sidx.at[j]], gbuf, gsem).wait()
        pltpu.sync_copy(gbuf, acc.at[didx.at[j]], add=True)
        return carry

    lax.fori_loop(0, cnt, body, 0)
    plsc.subcore_barrier()
    pltpu.sync_copy(acc.at[pl.ds(sid * ROWS_PER_TILE, ROWS_PER_TILE)],
                    out_hbm.at[cid, pl.ds(sid * ROWS_PER_TILE, ROWS_PER_TILE)])


def _run_scatter(y, src3, dst3, zeros_big):
    if "scatter" not in _sc_kernel_cache:
        mesh = plsc.VectorSubcoreMesh(core_axis_name="c", subcore_axis_name="s")
        _sc_kernel_cache["scatter"] = functools.partial(
            pl.kernel,
            out_type=jax.ShapeDtypeStruct((NC, NPAD, D), jnp.float32),
            mesh=mesh,
            scratch_types=[
                pltpu.VMEM((KMAX, CHUNK), jnp.int32),
                pltpu.VMEM((KMAX, CHUNK), jnp.int32),
                pltpu.VMEM((CHUNK, D), jnp.float32),
                pltpu.VMEM_SHARED((NPAD, D), jnp.float32),
                pltpu.SemaphoreType.DMA,
            ],
        )(_sc_scatter_body)
    return _sc_kernel_cache["scatter"](y, src3, dst3, zeros_big)


def _prep_body(degp_ref, x_ref, w_ref, dis_ref, y_ref):
    deg = degp_ref[0, :N, 0:1] + degp_ref[1, :N, 0:1] + 1.0
    dis = lax.rsqrt(deg)
    dis_ref[...] = dis
    xw = jnp.dot(x_ref[...], w_ref[...], preferred_element_type=jnp.float32)
    y_ref[...] = dis * xw


_prep_kernel = pl.pallas_call(
    _prep_body,
    out_shape=(jax.ShapeDtypeStruct((N, 1), jnp.float32),
               jax.ShapeDtypeStruct((N, D), jnp.float32)),
)


def _bn_relu(zp_ref, y_ref, dis_ref, b_ref, g_ref, be_ref):
    z = zp_ref[0, :N, :] + zp_ref[1, :N, :]
    t = dis_ref[...] * (z + y_ref[...]) + b_ref[...]
    m = jnp.mean(t, axis=0, keepdims=True)
    v = jnp.mean((t - m) * (t - m), axis=0, keepdims=True)
    t = (t - m) * lax.rsqrt(v + 1e-5) * g_ref[...] + be_ref[...]
    return jnp.maximum(t, 0.0)


def _mid_even_body(zp_ref, y_ref, dis_ref, b_ref, g_ref, be_ref, wn_ref,
                   ynext_ref):
    t = _bn_relu(zp_ref, y_ref, dis_ref, b_ref, g_ref, be_ref)
    ynext_ref[...] = dis_ref[...] * jnp.dot(
        t, wn_ref[...], preferred_element_type=jnp.float32)


_mid_even_kernel = pl.pallas_call(
    _mid_even_body,
    out_shape=jax.ShapeDtypeStruct((N, D), jnp.float32),
)


def _mid_odd_body(zp_ref, y_ref, dis_ref, b_ref, g_ref, be_ref, hin_ref,
                  wn_ref, hnew_ref, ynext_ref):
    t = _bn_relu(zp_ref, y_ref, dis_ref, b_ref, g_ref, be_ref)
    h = (t + hin_ref[...]) * INV_SQRT2
    hnew_ref[...] = h
    ynext_ref[...] = dis_ref[...] * jnp.dot(
        h, wn_ref[...], preferred_element_type=jnp.float32)


_mid_odd_kernel = pl.pallas_call(
    _mid_odd_body,
    out_shape=(jax.ShapeDtypeStruct((N, D), jnp.float32),
               jax.ShapeDtypeStruct((N, D), jnp.float32)),
)


def _final_body(zp_ref, y_ref, dis_ref, b_ref, g_ref, be_ref, hin_ref,
                batch_ref, wh1_ref, bh1_ref, wh2_ref, bh2_ref, out_ref):
    t = _bn_relu(zp_ref, y_ref, dis_ref, b_ref, g_ref, be_ref)
    h = (t + hin_ref[...]) * INV_SQRT2
    seg = batch_ref[...]
    gid = lax.broadcasted_iota(jnp.int32, (1, NG), 1)
    p = (seg == gid).astype(jnp.float32)
    dn = (((0,), (0,)), ((), ()))
    sums = lax.dot_general(p, h, dn, preferred_element_type=jnp.float32)
    cnt = lax.dot_general(p, jnp.ones((N, 1), jnp.float32), dn,
                          preferred_element_type=jnp.float32)
    pooled = sums / jnp.maximum(cnt, 1.0)
    z1 = jnp.dot(pooled, wh1_ref[...], preferred_element_type=jnp.float32)
    z1 = jnp.maximum(z1 + bh1_ref[...], 0.0)
    z2 = jnp.dot(z1, wh2_ref[...], preferred_element_type=jnp.float32)
    out_ref[...] = z2 + bh2_ref[...]


_final_kernel = pl.pallas_call(
    _final_body,
    out_shape=jax.ShapeDtypeStruct((NG, NOUT), jnp.float32),
)


def kernel(x, edge_index, batch, W1s, b1s, g1s, be1s, W2s, b2s, g2s, be2s,
           Wh1, bh1, Wh2, bh2):
    src = edge_index[0]
    dst = edge_index[1]
    pad = EPAD - E
    srcp = jnp.concatenate([src, jnp.zeros((pad,), jnp.int32)])
    dstp = jnp.concatenate([dst, jnp.full((pad,), NPAD - 1, jnp.int32)])
    src3 = srcp.reshape(TOTC, CHUNK)
    dst3 = dstp.reshape(TOTC, CHUNK)

    zeros_big = jnp.zeros((NPAD, D), jnp.float32)
    ones_big = jnp.ones((CHUNK, D), jnp.float32)
    batch2d = batch.reshape(N, 1)

    ws = [W1s[0], W2s[0], W1s[1], W2s[1], W1s[2], W2s[2]]
    bs = [b1s[0], b2s[0], b1s[1], b2s[1], b1s[2], b2s[2]]
    gs = [g1s[0], g2s[0], g1s[1], g2s[1], g1s[2], g2s[2]]
    bes = [be1s[0], be2s[0], be1s[1], be2s[1], be1s[2], be2s[2]]
    bs = [b.reshape(1, D) for b in bs]
    gs = [g.reshape(1, D) for g in gs]
    bes = [b.reshape(1, D) for b in bes]

    degp = _run_deg(dst3, zeros_big, ones_big)
    dis, y = _prep_kernel(degp, x, ws[0])

    h_in = x
    out = None
    for i in range(6):
        zp = _run_scatter(y, src3, dst3, zeros_big)
        if i == 5:
            out = _final_kernel(zp, y, dis, bs[i], gs[i], bes[i], h_in,
                                batch2d, Wh1, bh1.reshape(1, NHID), Wh2,
                                bh2.reshape(1, NOUT))
        elif i % 2 == 0:
            y = _mid_even_kernel(zp, y, dis, bs[i], gs[i], bes[i], ws[i + 1])
        else:
            h_in, y = _mid_odd_kernel(zp, y, dis, bs[i], gs[i], bes[i], h_in,
                                      ws[i + 1])
    return out


# speedup vs baseline: 1.5124x; 1.3082x over previous
import functools

import jax
import jax.numpy as jnp
import numpy as np
from jax import lax
from jax.experimental import pallas as pl
from jax.experimental.pallas import tpu as pltpu
from jax.experimental.pallas import tpu_sc as plsc

N = 10000
E = 320000
D = 128
NG = 32
NHID = 256
NOUT = 768

NC = 2
NS = 16
NW = NC * NS
CHUNK = 128
K = -(-E // (NW * CHUNK))
EPAD = NW * K * CHUNK
NPAD = 10240
ROWS_PER_TILE = NPAD // NS
INV_SQRT2 = float(1.0 / np.sqrt(2.0))


_sc_kernel_cache = {}


def _sc_deg_body(dst_hbm, zeros_hbm, ones_hbm, out_hbm, didx, ones_v, acc):
    cid = lax.axis_index("c")
    sid = lax.axis_index("s")
    wid = cid * NS + sid
    pltpu.sync_copy(zeros_hbm.at[pl.ds(sid * ROWS_PER_TILE, ROWS_PER_TILE)],
                    acc.at[pl.ds(sid * ROWS_PER_TILE, ROWS_PER_TILE)])
    pltpu.sync_copy(dst_hbm.at[wid], didx)
    pltpu.sync_copy(ones_hbm, ones_v)
    plsc.subcore_barrier()

    def body(j, carry):
        pltpu.sync_copy(ones_v, acc.at[didx.at[j]], add=True)
        return carry

    lax.fori_loop(0, K, body, 0)
    plsc.subcore_barrier()
    pltpu.sync_copy(acc.at[pl.ds(sid * ROWS_PER_TILE, ROWS_PER_TILE)],
                    out_hbm.at[cid, pl.ds(sid * ROWS_PER_TILE, ROWS_PER_TILE)])


def _run_deg(dst3, zeros_big, ones_big):
    if "deg" not in _sc_kernel_cache:
        mesh = plsc.VectorSubcoreMesh(core_axis_name="c", subcore_axis_name="s")
        _sc_kernel_cache["deg"] = functools.partial(
            pl.kernel,
            out_type=jax.ShapeDtypeStruct((NC, NPAD, D), jnp.float32),
            mesh=mesh,
            scratch_types=[
                pltpu.VMEM((K, CHUNK), jnp.int32),
                pltpu.VMEM((CHUNK, D), jnp.float32),
                pltpu.VMEM_SHARED((NPAD, D), jnp.float32),
            ],
        )(_sc_deg_body)
    return _sc_kernel_cache["deg"](dst3, zeros_big, ones_big)


def _sc_scatter_body(y_hbm, src_hbm, dst_hbm, zeros_hbm, out_hbm,
                     sidx, didx, gbuf, acc, gsem):
    cid = lax.axis_index("c")
    sid = lax.axis_index("s")
    wid = cid * NS + sid
    pltpu.sync_copy(zeros_hbm.at[pl.ds(sid * ROWS_PER_TILE, ROWS_PER_TILE)],
                    acc.at[pl.ds(sid * ROWS_PER_TILE, ROWS_PER_TILE)])
    pltpu.sync_copy(src_hbm.at[wid], sidx)
    pltpu.sync_copy(dst_hbm.at[wid], didx)
    plsc.subcore_barrier()

    def body(j, carry):
        pltpu.async_copy(y_hbm.at[sidx.at[j]], gbuf, gsem).wait()
        pltpu.sync_copy(gbuf, acc.at[didx.at[j]], add=True)
        return carry

    lax.fori_loop(0, K, body, 0)
    plsc.subcore_barrier()
    pltpu.sync_copy(acc.at[pl.ds(sid * ROWS_PER_TILE, ROWS_PER_TILE)],
                    out_hbm.at[cid, pl.ds(sid * ROWS_PER_TILE, ROWS_PER_TILE)])


def _run_scatter(y, src3, dst3, zeros_big):
    if "scatter" not in _sc_kernel_cache:
        mesh = plsc.VectorSubcoreMesh(core_axis_name="c", subcore_axis_name="s")
        _sc_kernel_cache["scatter"] = functools.partial(
            pl.kernel,
            out_type=jax.ShapeDtypeStruct((NC, NPAD, D), jnp.float32),
            mesh=mesh,
            scratch_types=[
                pltpu.VMEM((K, CHUNK), jnp.int32),
                pltpu.VMEM((K, CHUNK), jnp.int32),
                pltpu.VMEM((CHUNK, D), jnp.float32),
                pltpu.VMEM_SHARED((NPAD, D), jnp.float32),
                pltpu.SemaphoreType.DMA,
            ],
        )(_sc_scatter_body)
    return _sc_kernel_cache["scatter"](y, src3, dst3, zeros_big)


def _prep_body(degp_ref, x_ref, w_ref, dis_ref, y_ref):
    deg = degp_ref[0, :N, 0:1] + degp_ref[1, :N, 0:1] + 1.0
    dis = lax.rsqrt(deg)
    dis_ref[...] = dis
    xw = jnp.dot(x_ref[...], w_ref[...], preferred_element_type=jnp.float32)
    y_ref[...] = dis * xw


_prep_kernel = pl.pallas_call(
    _prep_body,
    out_shape=(jax.ShapeDtypeStruct((N, 1), jnp.float32),
               jax.ShapeDtypeStruct((N, D), jnp.float32)),
)


def _bn_relu(zp_ref, y_ref, dis_ref, b_ref, g_ref, be_ref):
    z = zp_ref[0, :N, :] + zp_ref[1, :N, :]
    t = dis_ref[...] * (z + y_ref[...]) + b_ref[...]
    m = jnp.mean(t, axis=0, keepdims=True)
    v = jnp.mean((t - m) * (t - m), axis=0, keepdims=True)
    t = (t - m) * lax.rsqrt(v + 1e-5) * g_ref[...] + be_ref[...]
    return jnp.maximum(t, 0.0)


def _mid_even_body(zp_ref, y_ref, dis_ref, b_ref, g_ref, be_ref, wn_ref,
                   ynext_ref):
    t = _bn_relu(zp_ref, y_ref, dis_ref, b_ref, g_ref, be_ref)
    ynext_ref[...] = dis_ref[...] * jnp.dot(
        t, wn_ref[...], preferred_element_type=jnp.float32)


_mid_even_kernel = pl.pallas_call(
    _mid_even_body,
    out_shape=jax.ShapeDtypeStruct((N, D), jnp.float32),
)


def _mid_odd_body(zp_ref, y_ref, dis_ref, b_ref, g_ref, be_ref, hin_ref,
                  wn_ref, hnew_ref, ynext_ref):
    t = _bn_relu(zp_ref, y_ref, dis_ref, b_ref, g_ref, be_ref)
    h = (t + hin_ref[...]) * INV_SQRT2
    hnew_ref[...] = h
    ynext_ref[...] = dis_ref[...] * jnp.dot(
        h, wn_ref[...], preferred_element_type=jnp.float32)


_mid_odd_kernel = pl.pallas_call(
    _mid_odd_body,
    out_shape=(jax.ShapeDtypeStruct((N, D), jnp.float32),
               jax.ShapeDtypeStruct((N, D), jnp.float32)),
)


def _final_body(zp_ref, y_ref, dis_ref, b_ref, g_ref, be_ref, hin_ref,
                batch_ref, wh1_ref, bh1_ref, wh2_ref, bh2_ref, out_ref):
    t = _bn_relu(zp_ref, y_ref, dis_ref, b_ref, g_ref, be_ref)
    h = (t + hin_ref[...]) * INV_SQRT2
    seg = batch_ref[...]
    gid = lax.broadcasted_iota(jnp.int32, (1, NG), 1)
    p = (seg == gid).astype(jnp.float32)
    dn = (((0,), (0,)), ((), ()))
    sums = lax.dot_general(p, h, dn, preferred_element_type=jnp.float32)
    cnt = lax.dot_general(p, jnp.ones((N, 1), jnp.float32), dn,
                          preferred_element_type=jnp.float32)
    pooled = sums / jnp.maximum(cnt, 1.0)
    z1 = jnp.dot(pooled, wh1_ref[...], preferred_element_type=jnp.float32)
    z1 = jnp.maximum(z1 + bh1_ref[...], 0.0)
    z2 = jnp.dot(z1, wh2_ref[...], preferred_element_type=jnp.float32)
    out_ref[...] = z2 + bh2_ref[...]


_final_kernel = pl.pallas_call(
    _final_body,
    out_shape=jax.ShapeDtypeStruct((NG, NOUT), jnp.float32),
)


def kernel(x, edge_index, batch, W1s, b1s, g1s, be1s, W2s, b2s, g2s, be2s,
           Wh1, bh1, Wh2, bh2):
    src = edge_index[0]
    dst = edge_index[1]
    pad = EPAD - E
    srcp = jnp.concatenate([src, jnp.zeros((pad,), jnp.int32)])
    dstp = jnp.concatenate([dst, jnp.full((pad,), NPAD - 1, jnp.int32)])
    src3 = srcp.reshape(NW, K, CHUNK)
    dst3 = dstp.reshape(NW, K, CHUNK)

    zeros_big = jnp.zeros((NPAD, D), jnp.float32)
    ones_big = jnp.ones((CHUNK, D), jnp.float32)
    batch2d = batch.reshape(N, 1)

    ws = [W1s[0], W2s[0], W1s[1], W2s[1], W1s[2], W2s[2]]
    bs = [b1s[0], b2s[0], b1s[1], b2s[1], b1s[2], b2s[2]]
    gs = [g1s[0], g2s[0], g1s[1], g2s[1], g1s[2], g2s[2]]
    bes = [be1s[0], be2s[0], be1s[1], be2s[1], be1s[2], be2s[2]]
    bs = [b.reshape(1, D) for b in bs]
    gs = [g.reshape(1, D) for g in gs]
    bes = [b.reshape(1, D) for b in bes]

    degp = _run_deg(dst3, zeros_big, ones_big)
    dis, y = _prep_kernel(degp, x, ws[0])

    h_in = x
    out = None
    for i in range(6):
        zp = _run_scatter(y, src3, dst3, zeros_big)
        if i == 5:
            out = _final_kernel(zp, y, dis, bs[i], gs[i], bes[i], h_in,
                                batch2d, Wh1, bh1.reshape(1, NHID), Wh2,
                                bh2.reshape(1, NOUT))
        elif i % 2 == 0:
            y = _mid_even_kernel(zp, y, dis, bs[i], gs[i], bes[i], ws[i + 1])
        else:
            h_in, y = _mid_odd_kernel(zp, y, dis, bs[i], gs[i], bes[i], h_in,
                                      ws[i + 1])
    return out
